# baseline (speedup 1.0000x reference)
import functools
import math

import jax
import jax.numpy as jnp
from jax import lax
from jax.experimental import pallas as pl
from jax.experimental.pallas import tpu as pltpu
from jax.experimental.pallas import tpu_sc as plsc

_N = 10000
_E = 320000
_D = 128
_ED = 16
_C = 128

_NC = 2
_NS = 16
_NW = _NC * _NS
_EW = _E // _NW
_K = 80
_STEPS = _EW // _K

_NSEG = 5
_ES = _E // _NSEG
_EWS = _ES // _NW
_SSTEPS = _EWS // _K

_BE = 2000
_GSTEPS = _ES // _BE

_f32 = jnp.float32


_bf16 = jnp.bfloat16
_PI = jnp.asarray(list(range(0, _C, 2)) + list(range(1, _C, 2)), jnp.int32)


def _tables_body(x_ref, wd_ref, bd_ref, ws_ref, bs_ref, td_ref, ts_ref):
    xx = x_ref[...]
    td_ref[...] = jnp.dot(xx, wd_ref[...], preferred_element_type=_f32) + bd_ref[...]
    ts_ref[...] = jnp.dot(xx, ws_ref[...], preferred_element_type=_f32) + bs_ref[...]


def _node_tables(x, wd, bd, ws, bs):
    return pl.pallas_call(
        _tables_body,
        out_shape=[
            jax.ShapeDtypeStruct((_N, 3 * _C), _f32),
            jax.ShapeDtypeStruct((_N, 2 * _C), _f32),
        ],
    )(x, wd, bd, ws, bs)


def _pack_i32(t, pad_to=None):
    n, w = t.shape
    if pad_to is not None and pad_to > w:
        t = jnp.concatenate([t, jnp.zeros((n, pad_to - w), t.dtype)], axis=1)
        w = pad_to
    return lax.bitcast_convert_type(
        t.astype(_bf16).reshape(n, w // 2, 2), jnp.int32)


def _unpack(x):
    lo = lax.bitcast_convert_type(x << 16, _f32)
    hi = lax.bitcast_convert_type(
        x & jnp.int32(-65536), _f32)
    return jnp.concatenate([lo, hi], axis=-1)


def _gather_sc_body(seg, td_hbm, ts_hbm, dst_hbm, src_hbm, gd_hbm, gs_hbm,
                    dsta, srca, gdv0, gsv0, gdv1, gsv1,
                    semg0, semg1, semo0, semo1):
    wid = lax.axis_index("s") * _NC + lax.axis_index("c")
    base0 = pl.multiple_of(wid * _EWS, 8)
    ibase0 = pl.multiple_of(seg * _ES + wid * _EWS, 8)

    pltpu.sync_copy(dst_hbm.at[pl.ds(ibase0, _EWS)], dsta)
    pltpu.sync_copy(src_hbm.at[pl.ds(ibase0, _EWS)], srca)

    sets = ((gdv0, gsv0, semg0, semo0),
            (gdv1, gsv1, semg1, semo1))

    def wait_out(st):
        gdv, gsv, semg, semo = st
        pltpu.make_async_copy(gdv, gd_hbm.at[pl.ds(base0, _K)], semo).wait()
        pltpu.make_async_copy(gsv, gs_hbm.at[pl.ds(base0, _K)], semo).wait()

    def fire(c, st):
        gdv, gsv, semg, semo = st
        off = pl.multiple_of(c * _K, 8)
        pltpu.async_copy(td_hbm.at[dsta.at[pl.ds(off, _K)]], gdv, semg)
        pltpu.async_copy(ts_hbm.at[srca.at[pl.ds(off, _K)]], gsv, semg)

    def complete(c, st):
        gdv, gsv, semg, semo = st
        base = pl.multiple_of(base0 + c * _K, 8)
        pltpu.make_async_copy(td_hbm.at[dsta.at[pl.ds(0, _K)]], gdv, semg).wait()
        pltpu.make_async_copy(ts_hbm.at[srca.at[pl.ds(0, _K)]], gsv, semg).wait()
        pltpu.async_copy(gdv, gd_hbm.at[pl.ds(base, _K)], semo)
        pltpu.async_copy(gsv, gs_hbm.at[pl.ds(base, _K)], semo)

    def step(g, carry):
        for par in (0, 1):
            st = sets[par]

            @pl.when(jnp.logical_and(g % 2 == par, g < _SSTEPS))
            def _(st=st):
                @pl.when(g >= 2)
                def _w():
                    wait_out(st)
                fire(g, st)
        for par in (0, 1):
            st = sets[par]

            @pl.when(jnp.logical_and((g - 1) % 2 == par, g >= 1))
            def _(st=st):
                complete(g - 1, st)
        return carry

    lax.fori_loop(0, _SSTEPS + 1, step, 0)
    for st in sets:
        wait_out(st)


def _gather(seg, td, ts, dst, src):
    fn = pl.kernel(
        functools.partial(_gather_sc_body, seg),
        out_type=[
            jax.ShapeDtypeStruct((_ES, 2 * _C), jnp.int32),
            jax.ShapeDtypeStruct((_ES, _C), jnp.int32),
        ],
        mesh=plsc.VectorSubcoreMesh(core_axis_name="c", subcore_axis_name="s"),
        scratch_types=[
            pltpu.VMEM((_EWS,), jnp.int32),
            pltpu.VMEM((_EWS,), jnp.int32),
            pltpu.VMEM((_K, 2 * _C), jnp.int32),
            pltpu.VMEM((_K, _C), jnp.int32),
            pltpu.VMEM((_K, 2 * _C), jnp.int32),
            pltpu.VMEM((_K, _C), jnp.int32),
            pltpu.SemaphoreType.DMA,
            pltpu.SemaphoreType.DMA,
            pltpu.SemaphoreType.DMA,
            pltpu.SemaphoreType.DMA,
        ],
    )
    return fn(td, ts, dst, src)


def _pass1a_body(ea_ref, gd_ref, gs_ref, wek_ref, wku2_ref,
                 bhk_ref, bku2_ref, alpha_ref, stats_ref):
    i = pl.program_id(0)
    ea = ea_ref[...]
    h = _C // 2
    gq = _unpack(gd_ref[:, :h])
    gka = _unpack(gd_ref[:, h:])
    gkb = _unpack(gs_ref[:, :h])
    hk = (gka + gkb
          + jnp.dot(ea, wek_ref[...], preferred_element_type=_f32) + bhk_ref[...])
    hk = hk * jax.nn.sigmoid(hk)
    kj = jnp.dot(hk, wku2_ref[...], preferred_element_type=_f32) + bku2_ref[...]
    alpha = gq * kj * (1.0 / math.sqrt(_C))
    alpha_ref[...] = alpha

    @pl.when(i == 0)
    def _():
        stats_ref[...] = jnp.zeros_like(stats_ref)

    stats_ref[0:1, :] += jnp.sum(alpha, axis=0, keepdims=True)
    stats_ref[1:2, :] += jnp.sum(alpha * alpha, axis=0, keepdims=True)


def _pass1a(seg, edge_attr, gd, gs, wek, wku2, bhk, bku2):
    full = lambda r, c: pl.BlockSpec((r, c), lambda i: (0, 0))
    return pl.pallas_call(
        _pass1a_body,
        grid=(_GSTEPS,),
        in_specs=[
            pl.BlockSpec((_BE, _ED), lambda i: (i + seg * _GSTEPS, 0)),
            pl.BlockSpec((_BE, _C), lambda i: (i, 0)),
            pl.BlockSpec((_BE, _C), lambda i: (i, 0)),
            full(_ED, _C), full(_C, _C), full(1, _C), full(1, _C),
        ],
        out_specs=[
            pl.BlockSpec((_BE, _C), lambda i: (i, 0)),
            pl.BlockSpec((8, _C), lambda i: (0, 0)),
        ],
        out_shape=[
            jax.ShapeDtypeStruct((_ES, _C), _f32),
            jax.ShapeDtypeStruct((8, _C), _f32),
        ],
    )(edge_attr, gd, gs, wek, wku2, bhk, bku2)


def _pass1b_body(ea_ref, gd_ref, gs_ref, alpha_ref, wem_ref, wm2_ref,
                 bhv_ref, bm2_ref, scale_ref, shift_ref, out_ref):
    ea = ea_ref[...]
    h = _C // 2
    gva = _unpack(gd_ref[:, :h])
    gvb = _unpack(gs_ref[:, h:])
    hv = (gva + gvb
          + jnp.dot(ea, wem_ref[...], preferred_element_type=_f32) + bhv_ref[...])
    hv = hv * jax.nn.sigmoid(hv)
    msg = jnp.dot(hv, wm2_ref[...], preferred_element_type=_f32) + bm2_ref[...]
    a = alpha_ref[...] * scale_ref[...] + shift_ref[...]
    out_ref[...] = msg * jax.nn.sigmoid(a)


def _pass1b(seg, edge_attr, gd, gs, alpha, wem, wm2, bhv, bm2, scale, shift):
    full = lambda r, c: pl.BlockSpec((r, c), lambda i: (0, 0))
    return pl.pallas_call(
        _pass1b_body,
        grid=(_GSTEPS,),
        in_specs=[
            pl.BlockSpec((_BE, _ED), lambda i: (i + seg * _GSTEPS, 0)),
            pl.BlockSpec((_BE, _C), lambda i: (i, 1)),
            pl.BlockSpec((_BE, _C), lambda i: (i, 0)),
            pl.BlockSpec((_BE, _C), lambda i: (i, 0)),
            full(_ED, _C), full(_C, _C),
            full(1, _C), full(1, _C), full(1, _C), full(1, _C),
        ],
        out_specs=pl.BlockSpec((_BE, _C), lambda i: (i, 0)),
        out_shape=jax.ShapeDtypeStruct((_ES, _C), _f32),
    )(edge_attr, gd, gs, alpha, wem, wm2, bhv, bm2, scale, shift)


_RZ = 80
_NCH = _N // _RZ


def _scatter_sc_body(g0, g1, g2, g3, g4, dst_hbm, out_hbm,
                     idxv0, rowsv0, idxv1, rowsv1, zbuf, agg_sh, semr0, semr1):
    gated_segs = (g0, g1, g2, g3, g4)
    c = lax.axis_index("c")
    s = lax.axis_index("s")

    def zrow(r, carry):
        def zcol(j, carry2):
            zbuf[r, pl.ds(j * 16, 16)] = jnp.zeros((16,), _f32)
            return carry2
        return lax.fori_loop(0, _C // 16, zcol, carry)

    lax.fori_loop(0, _RZ, zrow, 0)

    def zinit(t, carry):
        @pl.when(t % _NS == s)
        def _():
            pltpu.sync_copy(zbuf, agg_sh.at[pl.ds(pl.multiple_of(t * _RZ, 8), _RZ)])
        return carry

    lax.fori_loop(0, _NCH, zinit, 0)
    plsc.subcore_barrier()

    wbase = pl.multiple_of((c * _NS + s) * _EWS, 8)
    sets = ((idxv0, rowsv0, semr0), (idxv1, rowsv1, semr1))

    for seg in range(_NSEG):
        gated_hbm = gated_segs[seg]
        dbase0 = pl.multiple_of(seg * _ES + wbase, 8)

        def fire(i, st, gated_hbm=gated_hbm, dbase0=dbase0):
            idxv, rowsv, semr = st
            base = pl.multiple_of(wbase + i * _K, 8)
            dbase = pl.multiple_of(dbase0 + i * _K, 8)
            pltpu.async_copy(dst_hbm.at[pl.ds(dbase, _K)], idxv, semr)
            pltpu.async_copy(gated_hbm.at[pl.ds(base, _K)], rowsv, semr)

        def complete(i, st, gated_hbm=gated_hbm, dbase0=dbase0):
            idxv, rowsv, semr = st
            base = pl.multiple_of(wbase + i * _K, 8)
            dbase = pl.multiple_of(dbase0 + i * _K, 8)
            pltpu.make_async_copy(dst_hbm.at[pl.ds(dbase, _K)], idxv, semr).wait()
            pltpu.make_async_copy(gated_hbm.at[pl.ds(base, _K)], rowsv, semr).wait()
            pltpu.sync_copy(rowsv, agg_sh.at[idxv], add=True)

        def step(g, carry, fire=fire, complete=complete):
            for par in (0, 1):
                st = sets[par]

                @pl.when(jnp.logical_and(g % 2 == par, g < _SSTEPS))
                def _(st=st):
                    fire(g, st)
            for par in (0, 1):
                st = sets[par]

                @pl.when(jnp.logical_and((g - 1) % 2 == par, g >= 1))
                def _(st=st):
                    complete(g - 1, st)
            return carry

        lax.fori_loop(0, _SSTEPS + 1, step, 0)
    plsc.subcore_barrier()

    def drain(t, carry):
        @pl.when(t % _NS == s)
        def _():
            off = pl.multiple_of(t * _RZ, 8)
            pltpu.sync_copy(agg_sh.at[pl.ds(off, _RZ)], zbuf)
            pltpu.sync_copy(zbuf, out_hbm.at[c, pl.ds(off, _RZ)])
        return carry

    lax.fori_loop(0, _NCH, drain, 0)


def _scatter(gated_segs, dst):
    fn = pl.kernel(
        _scatter_sc_body,
        out_type=jax.ShapeDtypeStruct((_NC, _N, _C), _f32),
        mesh=plsc.VectorSubcoreMesh(core_axis_name="c", subcore_axis_name="s"),
        scratch_types=[
            pltpu.VMEM((_K,), jnp.int32),
            pltpu.VMEM((_K, _C), _f32),
            pltpu.VMEM((_K,), jnp.int32),
            pltpu.VMEM((_K, _C), _f32),
            pltpu.VMEM((_RZ, _C), _f32),
            pltpu.VMEM_SHARED((_N, _C), _f32),
            pltpu.SemaphoreType.DMA,
            pltpu.SemaphoreType.DMA,
        ],
    )
    return fn(*gated_segs, dst)


def _final_body(parts_ref, x_ref, wc_ref, bc_ref, g_ref, b_ref, out_ref):
    agg = parts_ref[0] + parts_ref[1]
    out = jnp.dot(agg, wc_ref[...], preferred_element_type=_f32) + bc_ref[...]
    mu = jnp.mean(out, axis=0, keepdims=True)
    var = jnp.mean(out * out, axis=0, keepdims=True) - mu * mu
    out = (out - mu) / jnp.sqrt(var + 1e-5) * g_ref[...] + b_ref[...]
    out_ref[...] = jax.nn.softplus(x_ref[...] + out)


def _finalize(parts, x, wc, bc, g, b):
    return pl.pallas_call(
        _final_body,
        out_shape=jax.ShapeDtypeStruct((_N, _C), _f32),
    )(parts, x, wc, bc, g, b)


def kernel(x, edge_index, edge_attr, params):
    p = params
    src = edge_index[0].astype(jnp.int32)
    dst = edge_index[1].astype(jnp.int32)

    wku1a, wku1b, wku1c = p['Wku1'][:_C], p['Wku1'][_C:2 * _C], p['Wku1'][2 * _C:]
    wm1a, wm1b, wm1c = p['Wm1'][:_C], p['Wm1'][_C:2 * _C], p['Wm1'][2 * _C:]
    wd = jnp.concatenate([p['Wq'], p['Wk'] @ wku1a, p['Wv'] @ wm1a], axis=1)
    bd = jnp.concatenate([p['bq'], p['bk'] @ wku1a, p['bv'] @ wm1a]).reshape(1, -1)
    ws = jnp.concatenate([p['Wk'] @ wku1b, p['Wv'] @ wm1b], axis=1)
    bs = jnp.concatenate([p['bk'] @ wku1b, p['bv'] @ wm1b]).reshape(1, -1)
    pi = _PI
    wek = (p['We'] @ wku1c)[:, pi]
    wem = (p['We'] @ wm1c)[:, pi]
    bhk = (p['be'] @ wku1c + p['bku1'])[pi].reshape(1, -1)
    bhv = (p['be'] @ wm1c + p['bm1'])[pi].reshape(1, -1)
    wku2 = p['Wku2'][pi][:, pi]
    bku2 = p['bku2'][pi]
    wm2 = p['Wm2'][pi][:, pi]
    bm2 = p['bm2'][pi]
    g_att = p['g_att'][pi]
    b_att = p['b_att'][pi]
    wc = p['Wc'][pi, :]

    td, ts = _node_tables(x, wd, bd, ws, bs)
    td = _pack_i32(td, pad_to=4 * _C)
    ts = _pack_i32(ts)

    gathered, alphas, stats_l = [], [], []
    for sgm in range(_NSEG):
        gathered.append(_gather(sgm, td, ts, dst, src))
    for sgm in range(_NSEG):
        gd, gs = gathered[sgm]
        alpha, stats = _pass1a(sgm, edge_attr, gd, gs, wek, wku2,
                               bhk, bku2.reshape(1, -1))
        alphas.append(alpha)
        stats_l.append(stats)
    stats = sum(stats_l[1:], stats_l[0])
    mu = stats[0] / _E
    var = stats[1] / _E - mu * mu
    scale = g_att / jnp.sqrt(var + 1e-5)
    shift = b_att - mu * scale
    gateds = []
    for sgm in range(_NSEG):
        gd, gs = gathered[sgm]
        gateds.append(_pass1b(sgm, edge_attr, gd, gs, alphas[sgm], wem,
                              wm2, bhv, bm2.reshape(1, -1),
                              scale.reshape(1, -1), shift.reshape(1, -1)))
    parts = _scatter(gateds, dst)
    return _finalize(parts, x, wc, p['bc'].reshape(1, -1),
                     p['g_bn'].reshape(1, -1), p['b_bn'].reshape(1, -1))


# speedup vs baseline: 1.2390x; 1.2390x over previous
import functools
import math

import jax
import jax.numpy as jnp
from jax import lax
from jax.experimental import pallas as pl
from jax.experimental.pallas import tpu as pltpu
from jax.experimental.pallas import tpu_sc as plsc

_N = 10000
_E = 320000
_D = 128
_ED = 16
_C = 128

_NC = 2
_NS = 16
_NW = _NC * _NS
_EW = _E // _NW
_K = 80
_STEPS = _EW // _K

_NSEG = 5
_ES = _E // _NSEG
_EWS = _ES // _NW
_SSTEPS = _EWS // _K

_BE = 2000
_GSTEPS = _ES // _BE

_f32 = jnp.float32


_bf16 = jnp.bfloat16
_PI = jnp.asarray(list(range(0, _C, 2)) + list(range(1, _C, 2)), jnp.int32)


def _pack_block(m):
    h = _C // 2
    li = lax.bitcast_convert_type(m[:, :h].astype(_bf16).astype(_f32),
                                  jnp.int32)
    hi = lax.bitcast_convert_type(m[:, h:].astype(_bf16).astype(_f32),
                                  jnp.int32)
    return lax.shift_right_logical(li, 16) | (hi & jnp.int32(-65536))


def _tables_body(x_ref, wd_ref, bd_ref, ws_ref, bs_ref, td_ref, ts_ref):
    xx = x_ref[...]
    td = jnp.dot(xx, wd_ref[...], preferred_element_type=_f32) + bd_ref[...]
    ts = jnp.dot(xx, ws_ref[...], preferred_element_type=_f32) + bs_ref[...]
    h = _C // 2
    td_ref[:, :h] = _pack_block(td[:, :_C])
    td_ref[:, h:2 * h] = _pack_block(td[:, _C:2 * _C])
    td_ref[:, 2 * h:3 * h] = _pack_block(td[:, 2 * _C:])
    td_ref[:, 3 * h:] = jnp.zeros((xx.shape[0], h), jnp.int32)
    ts_ref[:, :h] = _pack_block(ts[:, :_C])
    ts_ref[:, h:] = _pack_block(ts[:, _C:])


def _node_tables(x, wd, bd, ws, bs):
    return pl.pallas_call(
        _tables_body,
        out_shape=[
            jax.ShapeDtypeStruct((_N, 2 * _C), jnp.int32),
            jax.ShapeDtypeStruct((_N, _C), jnp.int32),
        ],
    )(x, wd, bd, ws, bs)


def _unpack(x):
    lo = lax.bitcast_convert_type(x << 16, _f32)
    hi = lax.bitcast_convert_type(
        x & jnp.int32(-65536), _f32)
    return jnp.concatenate([lo, hi], axis=-1)


def _gather_sc_body(seg, td_hbm, ts_hbm, dst_hbm, src_hbm, gd_hbm, gs_hbm,
                    dsta, srca, gdv0, gsv0, gdv1, gsv1,
                    semg0, semg1, semo0, semo1):
    wid = lax.axis_index("s") * _NC + lax.axis_index("c")
    base0 = pl.multiple_of(wid * _EWS, 8)
    ibase0 = pl.multiple_of(seg * _ES + wid * _EWS, 8)

    pltpu.sync_copy(dst_hbm.at[pl.ds(ibase0, _EWS)], dsta)
    pltpu.sync_copy(src_hbm.at[pl.ds(ibase0, _EWS)], srca)

    sets = ((gdv0, gsv0, semg0, semo0),
            (gdv1, gsv1, semg1, semo1))

    def wait_out(st):
        gdv, gsv, semg, semo = st
        pltpu.make_async_copy(gdv, gd_hbm.at[pl.ds(base0, _K)], semo).wait()
        pltpu.make_async_copy(gsv, gs_hbm.at[pl.ds(base0, _K)], semo).wait()

    def fire(c, st):
        gdv, gsv, semg, semo = st
        off = pl.multiple_of(c * _K, 8)
        pltpu.async_copy(td_hbm.at[dsta.at[pl.ds(off, _K)]], gdv, semg)
        pltpu.async_copy(ts_hbm.at[srca.at[pl.ds(off, _K)]], gsv, semg)

    def complete(c, st):
        gdv, gsv, semg, semo = st
        base = pl.multiple_of(base0 + c * _K, 8)
        pltpu.make_async_copy(td_hbm.at[dsta.at[pl.ds(0, _K)]], gdv, semg).wait()
        pltpu.make_async_copy(ts_hbm.at[srca.at[pl.ds(0, _K)]], gsv, semg).wait()
        pltpu.async_copy(gdv, gd_hbm.at[pl.ds(base, _K)], semo)
        pltpu.async_copy(gsv, gs_hbm.at[pl.ds(base, _K)], semo)

    def step(g, carry):
        for par in (0, 1):
            st = sets[par]

            @pl.when(jnp.logical_and(g % 2 == par, g < _SSTEPS))
            def _(st=st):
                @pl.when(g >= 2)
                def _w():
                    wait_out(st)
                fire(g, st)
        for par in (0, 1):
            st = sets[par]

            @pl.when(jnp.logical_and((g - 1) % 2 == par, g >= 1))
            def _(st=st):
                complete(g - 1, st)
        return carry

    lax.fori_loop(0, _SSTEPS + 1, step, 0)
    for st in sets:
        wait_out(st)


def _gather(seg, td, ts, dst, src):
    fn = pl.kernel(
        functools.partial(_gather_sc_body, seg),
        out_type=[
            jax.ShapeDtypeStruct((_ES, 2 * _C), jnp.int32),
            jax.ShapeDtypeStruct((_ES, _C), jnp.int32),
        ],
        mesh=plsc.VectorSubcoreMesh(core_axis_name="c", subcore_axis_name="s"),
        scratch_types=[
            pltpu.VMEM((_EWS,), jnp.int32),
            pltpu.VMEM((_EWS,), jnp.int32),
            pltpu.VMEM((_K, 2 * _C), jnp.int32),
            pltpu.VMEM((_K, _C), jnp.int32),
            pltpu.VMEM((_K, 2 * _C), jnp.int32),
            pltpu.VMEM((_K, _C), jnp.int32),
            pltpu.SemaphoreType.DMA,
            pltpu.SemaphoreType.DMA,
            pltpu.SemaphoreType.DMA,
            pltpu.SemaphoreType.DMA,
        ],
    )
    return fn(td, ts, dst, src)


def _pass1a_body(ea_ref, gd_ref, gs_ref, wek_ref, wku2_ref,
                 bhk_ref, bku2_ref, alpha_ref, stats_ref):
    i = pl.program_id(0)
    ea = ea_ref[...]
    h = _C // 2
    gq = _unpack(gd_ref[:, :h])
    gka = _unpack(gd_ref[:, h:])
    gkb = _unpack(gs_ref[:, :h])
    hk = (gka + gkb
          + jnp.dot(ea, wek_ref[...], preferred_element_type=_f32) + bhk_ref[...])
    hk = hk * jax.nn.sigmoid(hk)
    kj = jnp.dot(hk, wku2_ref[...], preferred_element_type=_f32) + bku2_ref[...]
    alpha = gq * kj * (1.0 / math.sqrt(_C))
    alpha_ref[...] = alpha

    @pl.when(i == 0)
    def _():
        stats_ref[...] = jnp.zeros_like(stats_ref)

    stats_ref[0:1, :] += jnp.sum(alpha, axis=0, keepdims=True)
    stats_ref[1:2, :] += jnp.sum(alpha * alpha, axis=0, keepdims=True)


def _pass1a(seg, edge_attr, gd, gs, wek, wku2, bhk, bku2):
    full = lambda r, c: pl.BlockSpec((r, c), lambda i: (0, 0))
    return pl.pallas_call(
        _pass1a_body,
        grid=(_GSTEPS,),
        in_specs=[
            pl.BlockSpec((_BE, _ED), lambda i: (i + seg * _GSTEPS, 0)),
            pl.BlockSpec((_BE, _C), lambda i: (i, 0)),
            pl.BlockSpec((_BE, _C), lambda i: (i, 0)),
            full(_ED, _C), full(_C, _C), full(1, _C), full(1, _C),
        ],
        out_specs=[
            pl.BlockSpec((_BE, _C), lambda i: (i, 0)),
            pl.BlockSpec((8, _C), lambda i: (0, 0)),
        ],
        out_shape=[
            jax.ShapeDtypeStruct((_ES, _C), _f32),
            jax.ShapeDtypeStruct((8, _C), _f32),
        ],
    )(edge_attr, gd, gs, wek, wku2, bhk, bku2)


def _pass1b_body(ea_ref, gd_ref, gs_ref, alpha_ref, wem_ref, wm2_ref,
                 bhv_ref, bm2_ref, scale_ref, shift_ref, out_ref):
    ea = ea_ref[...]
    h = _C // 2
    gva = _unpack(gd_ref[:, :h])
    gvb = _unpack(gs_ref[:, h:])
    hv = (gva + gvb
          + jnp.dot(ea, wem_ref[...], preferred_element_type=_f32) + bhv_ref[...])
    hv = hv * jax.nn.sigmoid(hv)
    msg = jnp.dot(hv, wm2_ref[...], preferred_element_type=_f32) + bm2_ref[...]
    a = alpha_ref[...] * scale_ref[...] + shift_ref[...]
    out_ref[...] = msg * jax.nn.sigmoid(a)


def _pass1b(seg, edge_attr, gd, gs, alpha, wem, wm2, bhv, bm2, scale, shift):
    full = lambda r, c: pl.BlockSpec((r, c), lambda i: (0, 0))
    return pl.pallas_call(
        _pass1b_body,
        grid=(_GSTEPS,),
        in_specs=[
            pl.BlockSpec((_BE, _ED), lambda i: (i + seg * _GSTEPS, 0)),
            pl.BlockSpec((_BE, _C), lambda i: (i, 1)),
            pl.BlockSpec((_BE, _C), lambda i: (i, 0)),
            pl.BlockSpec((_BE, _C), lambda i: (i, 0)),
            full(_ED, _C), full(_C, _C),
            full(1, _C), full(1, _C), full(1, _C), full(1, _C),
        ],
        out_specs=pl.BlockSpec((_BE, _C), lambda i: (i, 0)),
        out_shape=jax.ShapeDtypeStruct((_ES, _C), _f32),
    )(edge_attr, gd, gs, alpha, wem, wm2, bhv, bm2, scale, shift)


_RZ = 80
_NCH = _N // _RZ


def _scatter_sc_body(g0, g1, g2, g3, g4, dst_hbm, out_hbm,
                     idxv0, rowsv0, idxv1, rowsv1, zbuf, agg_sh, semr0, semr1):
    gated_segs = (g0, g1, g2, g3, g4)
    c = lax.axis_index("c")
    s = lax.axis_index("s")

    def zrow(r, carry):
        def zcol(j, carry2):
            zbuf[r, pl.ds(j * 16, 16)] = jnp.zeros((16,), _f32)
            return carry2
        return lax.fori_loop(0, _C // 16, zcol, carry)

    lax.fori_loop(0, _RZ, zrow, 0)

    def zinit(t, carry):
        @pl.when(t % _NS == s)
        def _():
            pltpu.sync_copy(zbuf, agg_sh.at[pl.ds(pl.multiple_of(t * _RZ, 8), _RZ)])
        return carry

    lax.fori_loop(0, _NCH, zinit, 0)
    plsc.subcore_barrier()

    wbase = pl.multiple_of((c * _NS + s) * _EWS, 8)
    sets = ((idxv0, rowsv0, semr0), (idxv1, rowsv1, semr1))

    for seg in range(_NSEG):
        gated_hbm = gated_segs[seg]
        dbase0 = pl.multiple_of(seg * _ES + wbase, 8)

        def fire(i, st, gated_hbm=gated_hbm, dbase0=dbase0):
            idxv, rowsv, semr = st
            base = pl.multiple_of(wbase + i * _K, 8)
            dbase = pl.multiple_of(dbase0 + i * _K, 8)
            pltpu.async_copy(dst_hbm.at[pl.ds(dbase, _K)], idxv, semr)
            pltpu.async_copy(gated_hbm.at[pl.ds(base, _K)], rowsv, semr)

        def complete(i, st, gated_hbm=gated_hbm, dbase0=dbase0):
            idxv, rowsv, semr = st
            base = pl.multiple_of(wbase + i * _K, 8)
            dbase = pl.multiple_of(dbase0 + i * _K, 8)
            pltpu.make_async_copy(dst_hbm.at[pl.ds(dbase, _K)], idxv, semr).wait()
            pltpu.make_async_copy(gated_hbm.at[pl.ds(base, _K)], rowsv, semr).wait()
            pltpu.sync_copy(rowsv, agg_sh.at[idxv], add=True)

        def step(g, carry, fire=fire, complete=complete):
            for par in (0, 1):
                st = sets[par]

                @pl.when(jnp.logical_and(g % 2 == par, g < _SSTEPS))
                def _(st=st):
                    fire(g, st)
            for par in (0, 1):
                st = sets[par]

                @pl.when(jnp.logical_and((g - 1) % 2 == par, g >= 1))
                def _(st=st):
                    complete(g - 1, st)
            return carry

        lax.fori_loop(0, _SSTEPS + 1, step, 0)
    plsc.subcore_barrier()

    def drain(t, carry):
        @pl.when(t % _NS == s)
        def _():
            off = pl.multiple_of(t * _RZ, 8)
            pltpu.sync_copy(agg_sh.at[pl.ds(off, _RZ)], zbuf)
            pltpu.sync_copy(zbuf, out_hbm.at[c, pl.ds(off, _RZ)])
        return carry

    lax.fori_loop(0, _NCH, drain, 0)


def _scatter(gated_segs, dst):
    fn = pl.kernel(
        _scatter_sc_body,
        out_type=jax.ShapeDtypeStruct((_NC, _N, _C), _f32),
        mesh=plsc.VectorSubcoreMesh(core_axis_name="c", subcore_axis_name="s"),
        scratch_types=[
            pltpu.VMEM((_K,), jnp.int32),
            pltpu.VMEM((_K, _C), _f32),
            pltpu.VMEM((_K,), jnp.int32),
            pltpu.VMEM((_K, _C), _f32),
            pltpu.VMEM((_RZ, _C), _f32),
            pltpu.VMEM_SHARED((_N, _C), _f32),
            pltpu.SemaphoreType.DMA,
            pltpu.SemaphoreType.DMA,
        ],
    )
    return fn(*gated_segs, dst)


def _final_body(parts_ref, x_ref, wc_ref, bc_ref, g_ref, b_ref, out_ref):
    agg = parts_ref[0] + parts_ref[1]
    out = jnp.dot(agg, wc_ref[...], preferred_element_type=_f32) + bc_ref[...]
    mu = jnp.mean(out, axis=0, keepdims=True)
    var = jnp.mean(out * out, axis=0, keepdims=True) - mu * mu
    out = (out - mu) / jnp.sqrt(var + 1e-5) * g_ref[...] + b_ref[...]
    out_ref[...] = jax.nn.softplus(x_ref[...] + out)


def _finalize(parts, x, wc, bc, g, b):
    return pl.pallas_call(
        _final_body,
        out_shape=jax.ShapeDtypeStruct((_N, _C), _f32),
    )(parts, x, wc, bc, g, b)


def kernel(x, edge_index, edge_attr, params):
    p = params
    src = edge_index[0].astype(jnp.int32)
    dst = edge_index[1].astype(jnp.int32)

    pi0 = _PI
    wku1a, wku1b, wku1c = p['Wku1'][:_C], p['Wku1'][_C:2 * _C], p['Wku1'][2 * _C:]
    wm1a, wm1b, wm1c = p['Wm1'][:_C], p['Wm1'][_C:2 * _C], p['Wm1'][2 * _C:]
    wd = jnp.concatenate([p['Wq'][:, pi0], (p['Wk'] @ wku1a)[:, pi0],
                          (p['Wv'] @ wm1a)[:, pi0]], axis=1)
    bd = jnp.concatenate([p['bq'][pi0], (p['bk'] @ wku1a)[pi0],
                          (p['bv'] @ wm1a)[pi0]]).reshape(1, -1)
    ws = jnp.concatenate([(p['Wk'] @ wku1b)[:, pi0],
                          (p['Wv'] @ wm1b)[:, pi0]], axis=1)
    bs = jnp.concatenate([(p['bk'] @ wku1b)[pi0],
                          (p['bv'] @ wm1b)[pi0]]).reshape(1, -1)
    pi = _PI
    wek = (p['We'] @ wku1c)[:, pi]
    wem = (p['We'] @ wm1c)[:, pi]
    bhk = (p['be'] @ wku1c + p['bku1'])[pi].reshape(1, -1)
    bhv = (p['be'] @ wm1c + p['bm1'])[pi].reshape(1, -1)
    wku2 = p['Wku2'][pi][:, pi]
    bku2 = p['bku2'][pi]
    wm2 = p['Wm2'][pi][:, pi]
    bm2 = p['bm2'][pi]
    g_att = p['g_att'][pi]
    b_att = p['b_att'][pi]
    wc = p['Wc'][pi, :]

    td, ts = _node_tables(x, wd, bd, ws, bs)

    gathered, alphas, stats_l = [], [], []
    for sgm in range(_NSEG):
        gathered.append(_gather(sgm, td, ts, dst, src))
    for sgm in range(_NSEG):
        gd, gs = gathered[sgm]
        alpha, stats = _pass1a(sgm, edge_attr, gd, gs, wek, wku2,
                               bhk, bku2.reshape(1, -1))
        alphas.append(alpha)
        stats_l.append(stats)
    stats = sum(stats_l[1:], stats_l[0])
    mu = stats[0] / _E
    var = stats[1] / _E - mu * mu
    scale = g_att / jnp.sqrt(var + 1e-5)
    shift = b_att - mu * scale
    gateds = []
    for sgm in range(_NSEG):
        gd, gs = gathered[sgm]
        gateds.append(_pass1b(sgm, edge_attr, gd, gs, alphas[sgm], wem,
                              wm2, bhv, bm2.reshape(1, -1),
                              scale.reshape(1, -1), shift.reshape(1, -1)))
    parts = _scatter(gateds, dst)
    return _finalize(parts, x, wc, p['bc'].reshape(1, -1),
                     p['g_bn'].reshape(1, -1), p['b_bn'].reshape(1, -1))


# speedup vs baseline: 1.2641x; 1.0202x over previous
import functools
import math

import jax
import jax.numpy as jnp
from jax import lax
from jax.experimental import pallas as pl
from jax.experimental.pallas import tpu as pltpu
from jax.experimental.pallas import tpu_sc as plsc

_N = 10000
_E = 320000
_D = 128
_ED = 16
_C = 128

_NC = 2
_NS = 16
_NW = _NC * _NS
_EW = _E // _NW
_K = 80
_STEPS = _EW // _K

_NSEG = 5
_ES = _E // _NSEG
_EWS = _ES // _NW
_SSTEPS = _EWS // _K

_BE = 2000
_GSTEPS = _ES // _BE

_f32 = jnp.float32


_bf16 = jnp.bfloat16
_PI = jnp.asarray(list(range(0, _C, 2)) + list(range(1, _C, 2)), jnp.int32)


def _pack_block(m):
    h = _C // 2
    li = lax.bitcast_convert_type(m[:, :h].astype(_bf16).astype(_f32),
                                  jnp.int32)
    hi = lax.bitcast_convert_type(m[:, h:].astype(_bf16).astype(_f32),
                                  jnp.int32)
    return lax.shift_right_logical(li, 16) | (hi & jnp.int32(-65536))


def _tables_body(x_ref, wd_ref, bd_ref, ws_ref, bs_ref, td_ref, ts_ref):
    xx = x_ref[...]
    td = jnp.dot(xx, wd_ref[...], preferred_element_type=_f32) + bd_ref[...]
    ts = jnp.dot(xx, ws_ref[...], preferred_element_type=_f32) + bs_ref[...]
    h = _C // 2
    td_ref[:, :h] = _pack_block(td[:, :_C])
    td_ref[:, h:2 * h] = _pack_block(td[:, _C:2 * _C])
    td_ref[:, 2 * h:3 * h] = _pack_block(td[:, 2 * _C:])
    td_ref[:, 3 * h:] = jnp.zeros((xx.shape[0], h), jnp.int32)
    ts_ref[:, :h] = _pack_block(ts[:, :_C])
    ts_ref[:, h:] = _pack_block(ts[:, _C:])


def _node_tables(x, wd, bd, ws, bs):
    return pl.pallas_call(
        _tables_body,
        out_shape=[
            jax.ShapeDtypeStruct((_N, 2 * _C), jnp.int32),
            jax.ShapeDtypeStruct((_N, _C), jnp.int32),
        ],
    )(x, wd, bd, ws, bs)


def _unpack(x):
    lo = lax.bitcast_convert_type(x << 16, _f32)
    hi = lax.bitcast_convert_type(
        x & jnp.int32(-65536), _f32)
    return jnp.concatenate([lo, hi], axis=-1)


def _gather_sc_body(seg, td_hbm, ts_hbm, dst_hbm, src_hbm, gd_hbm, gs_hbm,
                    dsta, srca, gdv0, gsv0, gdv1, gsv1,
                    semg0, semg1, semo0, semo1):
    wid = lax.axis_index("s") * _NC + lax.axis_index("c")
    base0 = pl.multiple_of(wid * _EWS, 8)
    ibase0 = pl.multiple_of(seg * _ES + wid * _EWS, 8)

    pltpu.sync_copy(dst_hbm.at[pl.ds(ibase0, _EWS)], dsta)
    pltpu.sync_copy(src_hbm.at[pl.ds(ibase0, _EWS)], srca)

    sets = ((gdv0, gsv0, semg0, semo0),
            (gdv1, gsv1, semg1, semo1))

    def wait_out(st):
        gdv, gsv, semg, semo = st
        pltpu.make_async_copy(gdv, gd_hbm.at[pl.ds(base0, _K)], semo).wait()
        pltpu.make_async_copy(gsv, gs_hbm.at[pl.ds(base0, _K)], semo).wait()

    def fire(c, st):
        gdv, gsv, semg, semo = st
        off = pl.multiple_of(c * _K, 8)
        pltpu.async_copy(td_hbm.at[dsta.at[pl.ds(off, _K)]], gdv, semg)
        pltpu.async_copy(ts_hbm.at[srca.at[pl.ds(off, _K)]], gsv, semg)

    def complete(c, st):
        gdv, gsv, semg, semo = st
        base = pl.multiple_of(base0 + c * _K, 8)
        pltpu.make_async_copy(td_hbm.at[dsta.at[pl.ds(0, _K)]], gdv, semg).wait()
        pltpu.make_async_copy(ts_hbm.at[srca.at[pl.ds(0, _K)]], gsv, semg).wait()
        pltpu.async_copy(gdv, gd_hbm.at[pl.ds(base, _K)], semo)
        pltpu.async_copy(gsv, gs_hbm.at[pl.ds(base, _K)], semo)

    def step(g, carry):
        for par in (0, 1):
            st = sets[par]

            @pl.when(jnp.logical_and(g % 2 == par, g < _SSTEPS))
            def _(st=st):
                @pl.when(g >= 2)
                def _w():
                    wait_out(st)
                fire(g, st)
        for par in (0, 1):
            st = sets[par]

            @pl.when(jnp.logical_and((g - 1) % 2 == par, g >= 1))
            def _(st=st):
                complete(g - 1, st)
        return carry

    lax.fori_loop(0, _SSTEPS + 1, step, 0)
    for st in sets:
        wait_out(st)


def _gather(seg, td, ts, dst, src):
    fn = pl.kernel(
        functools.partial(_gather_sc_body, seg),
        out_type=[
            jax.ShapeDtypeStruct((_ES, 2 * _C), jnp.int32),
            jax.ShapeDtypeStruct((_ES, _C), jnp.int32),
        ],
        mesh=plsc.VectorSubcoreMesh(core_axis_name="c", subcore_axis_name="s"),
        scratch_types=[
            pltpu.VMEM((_EWS,), jnp.int32),
            pltpu.VMEM((_EWS,), jnp.int32),
            pltpu.VMEM((_K, 2 * _C), jnp.int32),
            pltpu.VMEM((_K, _C), jnp.int32),
            pltpu.VMEM((_K, 2 * _C), jnp.int32),
            pltpu.VMEM((_K, _C), jnp.int32),
            pltpu.SemaphoreType.DMA,
            pltpu.SemaphoreType.DMA,
            pltpu.SemaphoreType.DMA,
            pltpu.SemaphoreType.DMA,
        ],
    )
    return fn(td, ts, dst, src)


def _pass1a_body(ea_ref, gd_ref, gs_ref, wek_ref, wku2_ref,
                 bhk_ref, bku2_ref, alpha_ref, stats_ref):
    i = pl.program_id(0)
    ea = ea_ref[...]
    h = _C // 2
    gq = _unpack(gd_ref[:, :h])
    gka = _unpack(gd_ref[:, h:])
    gkb = _unpack(gs_ref[:, :h])
    hk = (gka + gkb
          + jnp.dot(ea, wek_ref[...], preferred_element_type=_f32) + bhk_ref[...])
    hk = hk * jax.nn.sigmoid(hk)
    kj = jnp.dot(hk, wku2_ref[...], preferred_element_type=_f32) + bku2_ref[...]
    alpha = gq * kj * (1.0 / math.sqrt(_C))
    alpha_ref[...] = alpha

    @pl.when(i == 0)
    def _():
        stats_ref[...] = jnp.zeros_like(stats_ref)

    stats_ref[0:1, :] += jnp.sum(alpha, axis=0, keepdims=True)
    stats_ref[1:2, :] += jnp.sum(alpha * alpha, axis=0, keepdims=True)


def _pass1a(seg, edge_attr, gd, gs, wek, wku2, bhk, bku2):
    full = lambda r, c: pl.BlockSpec((r, c), lambda i: (0, 0))
    return pl.pallas_call(
        _pass1a_body,
        grid=(_GSTEPS,),
        in_specs=[
            pl.BlockSpec((_BE, _ED), lambda i: (i + seg * _GSTEPS, 0)),
            pl.BlockSpec((_BE, _C), lambda i: (i, 0)),
            pl.BlockSpec((_BE, _C), lambda i: (i, 0)),
            full(_ED, _C), full(_C, _C), full(1, _C), full(1, _C),
        ],
        out_specs=[
            pl.BlockSpec((_BE, _C), lambda i: (i, 0)),
            pl.BlockSpec((8, _C), lambda i: (0, 0)),
        ],
        out_shape=[
            jax.ShapeDtypeStruct((_ES, _C), _f32),
            jax.ShapeDtypeStruct((8, _C), _f32),
        ],
    )(edge_attr, gd, gs, wek, wku2, bhk, bku2)


def _pass1b_body(ea_ref, gd_ref, gs_ref, alpha_ref, wem_ref, wm2_ref,
                 bhv_ref, bm2_ref, scale_ref, shift_ref, out_ref):
    ea = ea_ref[...]
    h = _C // 2
    gva = _unpack(gd_ref[:, :h])
    gvb = _unpack(gs_ref[:, h:])
    hv = (gva + gvb
          + jnp.dot(ea, wem_ref[...], preferred_element_type=_f32) + bhv_ref[...])
    hv = hv * jax.nn.sigmoid(hv)
    msg = jnp.dot(hv, wm2_ref[...], preferred_element_type=_f32) + bm2_ref[...]
    a = alpha_ref[...] * scale_ref[...] + shift_ref[...]
    out_ref[...] = msg * jax.nn.sigmoid(a)


def _pass1b(seg, edge_attr, gd, gs, alpha, wem, wm2, bhv, bm2, scale, shift):
    full = lambda r, c: pl.BlockSpec((r, c), lambda i: (0, 0))
    return pl.pallas_call(
        _pass1b_body,
        grid=(_GSTEPS,),
        in_specs=[
            pl.BlockSpec((_BE, _ED), lambda i: (i + seg * _GSTEPS, 0)),
            pl.BlockSpec((_BE, _C), lambda i: (i, 1)),
            pl.BlockSpec((_BE, _C), lambda i: (i, 0)),
            pl.BlockSpec((_BE, _C), lambda i: (i, 0)),
            full(_ED, _C), full(_C, _C),
            full(1, _C), full(1, _C), full(1, _C), full(1, _C),
        ],
        out_specs=pl.BlockSpec((_BE, _C), lambda i: (i, 0)),
        out_shape=jax.ShapeDtypeStruct((_ES, _C), _f32),
    )(edge_attr, gd, gs, alpha, wem, wm2, bhv, bm2, scale, shift)


_RZ = 80
_NCH = _N // _RZ


def _scatter_sc_body(segs, *refs):
    gated_segs = refs[:len(segs)]
    dst_hbm, out_hbm = refs[len(segs)], refs[len(segs) + 1]
    (idxv0, rowsv0, idxv1, rowsv1, zbuf, agg_sh,
     semr0, semr1) = refs[len(segs) + 2:]
    c = lax.axis_index("c")
    s = lax.axis_index("s")

    def zrow(r, carry):
        def zcol(j, carry2):
            zbuf[r, pl.ds(j * 16, 16)] = jnp.zeros((16,), _f32)
            return carry2
        return lax.fori_loop(0, _C // 16, zcol, carry)

    lax.fori_loop(0, _RZ, zrow, 0)

    def zinit(t, carry):
        @pl.when(t % _NS == s)
        def _():
            pltpu.sync_copy(zbuf, agg_sh.at[pl.ds(pl.multiple_of(t * _RZ, 8), _RZ)])
        return carry

    lax.fori_loop(0, _NCH, zinit, 0)
    plsc.subcore_barrier()

    wbase = pl.multiple_of((c * _NS + s) * _EWS, 8)
    sets = ((idxv0, rowsv0, semr0), (idxv1, rowsv1, semr1))

    for gated_hbm, seg in zip(gated_segs, segs):
        dbase0 = pl.multiple_of(seg * _ES + wbase, 8)

        def fire(i, st, gated_hbm=gated_hbm, dbase0=dbase0):
            idxv, rowsv, semr = st
            base = pl.multiple_of(wbase + i * _K, 8)
            dbase = pl.multiple_of(dbase0 + i * _K, 8)
            pltpu.async_copy(dst_hbm.at[pl.ds(dbase, _K)], idxv, semr)
            pltpu.async_copy(gated_hbm.at[pl.ds(base, _K)], rowsv, semr)

        def complete(i, st, gated_hbm=gated_hbm, dbase0=dbase0):
            idxv, rowsv, semr = st
            base = pl.multiple_of(wbase + i * _K, 8)
            dbase = pl.multiple_of(dbase0 + i * _K, 8)
            pltpu.make_async_copy(dst_hbm.at[pl.ds(dbase, _K)], idxv, semr).wait()
            pltpu.make_async_copy(gated_hbm.at[pl.ds(base, _K)], rowsv, semr).wait()
            pltpu.sync_copy(rowsv, agg_sh.at[idxv], add=True)

        def step(g, carry, fire=fire, complete=complete):
            for par in (0, 1):
                st = sets[par]

                @pl.when(jnp.logical_and(g % 2 == par, g < _SSTEPS))
                def _(st=st):
                    fire(g, st)
            for par in (0, 1):
                st = sets[par]

                @pl.when(jnp.logical_and((g - 1) % 2 == par, g >= 1))
                def _(st=st):
                    complete(g - 1, st)
            return carry

        lax.fori_loop(0, _SSTEPS + 1, step, 0)
    plsc.subcore_barrier()

    def drain(t, carry):
        @pl.when(t % _NS == s)
        def _():
            off = pl.multiple_of(t * _RZ, 8)
            pltpu.sync_copy(agg_sh.at[pl.ds(off, _RZ)], zbuf)
            pltpu.sync_copy(zbuf, out_hbm.at[c, pl.ds(off, _RZ)])
        return carry

    lax.fori_loop(0, _NCH, drain, 0)


def _scatter(gated_segs, segs, dst):
    fn = pl.kernel(
        functools.partial(_scatter_sc_body, segs),
        out_type=jax.ShapeDtypeStruct((_NC, _N, _C), _f32),
        mesh=plsc.VectorSubcoreMesh(core_axis_name="c", subcore_axis_name="s"),
        scratch_types=[
            pltpu.VMEM((_K,), jnp.int32),
            pltpu.VMEM((_K, _C), _f32),
            pltpu.VMEM((_K,), jnp.int32),
            pltpu.VMEM((_K, _C), _f32),
            pltpu.VMEM((_RZ, _C), _f32),
            pltpu.VMEM_SHARED((_N, _C), _f32),
            pltpu.SemaphoreType.DMA,
            pltpu.SemaphoreType.DMA,
        ],
    )
    return fn(*gated_segs, dst)


def _final_body(pa_ref, pb_ref, x_ref, wc_ref, bc_ref, g_ref, b_ref, out_ref):
    agg = pa_ref[0] + pa_ref[1] + pb_ref[0] + pb_ref[1]
    out = jnp.dot(agg, wc_ref[...], preferred_element_type=_f32) + bc_ref[...]
    mu = jnp.mean(out, axis=0, keepdims=True)
    var = jnp.mean(out * out, axis=0, keepdims=True) - mu * mu
    out = (out - mu) / jnp.sqrt(var + 1e-5) * g_ref[...] + b_ref[...]
    out_ref[...] = jax.nn.softplus(x_ref[...] + out)


def _finalize(parts_a, parts_b, x, wc, bc, g, b):
    return pl.pallas_call(
        _final_body,
        out_shape=jax.ShapeDtypeStruct((_N, _C), _f32),
    )(parts_a, parts_b, x, wc, bc, g, b)


def kernel(x, edge_index, edge_attr, params):
    p = params
    src = edge_index[0].astype(jnp.int32)
    dst = edge_index[1].astype(jnp.int32)

    pi0 = _PI
    wku1a, wku1b, wku1c = p['Wku1'][:_C], p['Wku1'][_C:2 * _C], p['Wku1'][2 * _C:]
    wm1a, wm1b, wm1c = p['Wm1'][:_C], p['Wm1'][_C:2 * _C], p['Wm1'][2 * _C:]
    wd = jnp.concatenate([p['Wq'][:, pi0], (p['Wk'] @ wku1a)[:, pi0],
                          (p['Wv'] @ wm1a)[:, pi0]], axis=1)
    bd = jnp.concatenate([p['bq'][pi0], (p['bk'] @ wku1a)[pi0],
                          (p['bv'] @ wm1a)[pi0]]).reshape(1, -1)
    ws = jnp.concatenate([(p['Wk'] @ wku1b)[:, pi0],
                          (p['Wv'] @ wm1b)[:, pi0]], axis=1)
    bs = jnp.concatenate([(p['bk'] @ wku1b)[pi0],
                          (p['bv'] @ wm1b)[pi0]]).reshape(1, -1)
    pi = _PI
    wek = (p['We'] @ wku1c)[:, pi]
    wem = (p['We'] @ wm1c)[:, pi]
    bhk = (p['be'] @ wku1c + p['bku1'])[pi].reshape(1, -1)
    bhv = (p['be'] @ wm1c + p['bm1'])[pi].reshape(1, -1)
    wku2 = p['Wku2'][pi][:, pi]
    bku2 = p['bku2'][pi]
    wm2 = p['Wm2'][pi][:, pi]
    bm2 = p['bm2'][pi]
    g_att = p['g_att'][pi]
    b_att = p['b_att'][pi]
    wc = p['Wc'][pi, :]

    td, ts = _node_tables(x, wd, bd, ws, bs)

    gathered, alphas, stats_l = [], [], []
    for sgm in range(_NSEG):
        gathered.append(_gather(sgm, td, ts, dst, src))
    for sgm in range(_NSEG):
        gd, gs = gathered[sgm]
        alpha, stats = _pass1a(sgm, edge_attr, gd, gs, wek, wku2,
                               bhk, bku2.reshape(1, -1))
        alphas.append(alpha)
        stats_l.append(stats)
    stats = sum(stats_l[1:], stats_l[0])
    mu = stats[0] / _E
    var = stats[1] / _E - mu * mu
    scale = g_att / jnp.sqrt(var + 1e-5)
    shift = b_att - mu * scale
    gateds = []
    for sgm in range(_NSEG):
        gd, gs = gathered[sgm]
        gateds.append(_pass1b(sgm, edge_attr, gd, gs, alphas[sgm], wem,
                              wm2, bhv, bm2.reshape(1, -1),
                              scale.reshape(1, -1), shift.reshape(1, -1)))
    parts_a = _scatter(gateds[:3], (0, 1, 2), dst)
    parts_b = _scatter(gateds[3:], (3, 4), dst)
    return _finalize(parts_a, parts_b, x, wc, p['bc'].reshape(1, -1),
                     p['g_bn'].reshape(1, -1), p['b_bn'].reshape(1, -1))
